# baseline (speedup 1.0000x reference)
import functools

import jax
import jax.numpy as jnp
import numpy as np
from jax import lax
from jax.experimental import pallas as pl
from jax.experimental.pallas import tpu as pltpu
from jax.experimental.pallas import tpu_sc as plsc

K = 256
NC, NS = 2, 16
NW = NC * NS


_INV_C = float(np.float32(1.0 / 768.0))


def _col_sum_768(v):
    total = None
    for w in range(3):
        p = v[256 * w: 256 * w + 128, :] + v[256 * w + 128: 256 * w + 256, :]
        t = p[0:8, :]
        for j in range(1, 16):
            t = t + p[8 * j: 8 * j + 8, :]
        u = t[0:4, :] + t[4:8, :]
        q = u[0:2, :] + u[2:4, :]
        r = q[0:1, :] + q[1:2, :]
        total = r if w == 0 else total + r
    return total


def _topk_body(x_ref, g_ref, b_ref, w_ref, cb_ref, gidx_ref, *, bid_off=0):
    n = x_ref.shape[1]
    c = x_ref.shape[2]
    bid = pl.program_id(0) + bid_off
    xt = jnp.transpose(x_ref[0])
    mu = _col_sum_768(xt) * _INV_C
    xc = xt - mu
    var = _col_sum_768(xc * xc) * _INV_C
    g_col = jnp.transpose(g_ref[...])
    b_col = jnp.transpose(b_ref[...])
    ln = xc / jnp.sqrt(var + 1e-5) * g_col + b_col
    wrep = jnp.broadcast_to(jnp.transpose(w_ref[...]), (c, 128))
    conf = lax.dot_general(ln.astype(jnp.bfloat16), wrep.astype(jnp.bfloat16),
                           (((0,), (0,)), ((), ())),
                           preferred_element_type=jnp.float32)[:, 0:1] + cb_ref[0, 0]
    m = jnp.max(conf)
    e = jnp.exp(conf - m)
    s_col = e / jnp.sum(e) * float(n)
    ii = lax.broadcasted_iota(jnp.int32, (n, n), 0)
    jj = lax.broadcasted_iota(jnp.int32, (n, n), 1)
    s_row = jnp.transpose(s_col)
    beats = (s_row > s_col) | ((s_row == s_col) & (jj < ii))
    rank = jnp.sum(beats.astype(jnp.int32), axis=1, keepdims=True)
    kk = lax.broadcasted_iota(jnp.int32, (n, K), 1)
    oh = rank == kk
    rows = lax.broadcasted_iota(jnp.int32, (n, K), 0)
    idx = jnp.sum(jnp.where(oh, rows, 0), axis=0, keepdims=True)
    gidx_ref[0] = idx + bid * n


_SC_CH = 32


def _sc_body(gidx_hbm, x_hbm, pos_hbm, pe_hbm, y_hbm, posd_hbm,
             idx_v, posd_v, ra0, rb0, ra1, rb1, sem_a, sem_b, sem_s0, sem_s1,
             *, nrows):
    wid = lax.axis_index("s") * NC + lax.axis_index("c")
    rows_per = nrows // NW
    base = wid * rows_per
    pltpu.sync_copy(gidx_hbm.at[pl.ds(base, rows_per)], idx_v)
    pltpu.async_copy(pos_hbm.at[idx_v], posd_v, sem_a).wait()
    pltpu.sync_copy(posd_v, posd_hbm.at[pl.ds(base, rows_per)])
    ra = (ra0, ra1)
    rb = (rb0, rb1)
    ss = (sem_s0, sem_s1)
    nch = rows_per // _SC_CH
    gathers = {}

    def _issue(c2):
        p2 = c2 & 1
        ia = idx_v.at[pl.ds(c2 * _SC_CH, _SC_CH)]
        pa = posd_v.at[pl.ds(c2 * _SC_CH, _SC_CH)]
        gathers[c2] = (pltpu.async_copy(x_hbm.at[ia], ra[p2], sem_a),
                       pltpu.async_copy(pe_hbm.at[pa], rb[p2], sem_b))

    scats = {}
    _issue(0)
    for cch in range(nch):
        p = cch & 1
        ga, gb = gathers.pop(cch)
        ga.wait()
        gb.wait()
        if cch + 1 < nch:
            if cch - 1 >= 0:
                scats.pop(cch - 1).wait()
            _issue(cch + 1)

        def _add_row(r, _, _p=p):
            for kk in range(768 // 16):
                sl = pl.ds(kk * 16, 16)
                ra[_p][r, sl] = ra[_p][r, sl] + rb[_p][r, sl]
            return 0

        lax.fori_loop(0, _SC_CH, _add_row, 0)
        scats[cch] = pltpu.async_copy(
            ra[p], y_hbm.at[pl.ds(base + cch * _SC_CH, _SC_CH)], ss[p])
    scats.pop(nch - 2).wait()
    scats.pop(nch - 1).wait()


def kernel(x, pos, pos_embed, H, W, ln_gamma, ln_beta, conf_w, conf_b):
    B, N, C = x.shape
    L = pos_embed.shape[1]
    x2d = x.reshape(B * N, C)
    pos_flat = pos.reshape(B * N)
    pe = pos_embed.reshape(L, C)
    mesh = plsc.VectorSubcoreMesh(core_axis_name="c", subcore_axis_name="s",
                                  num_cores=NC, num_subcores=NS)
    BH = B // 2
    ys, ps = [], []
    for h in range(2):
        gidx = pl.pallas_call(
            functools.partial(_topk_body, bid_off=h * BH),
            grid=(BH,),
            in_specs=[
                pl.BlockSpec((1, N, C), lambda b, _h=h: (b + _h * BH, 0, 0)),
                pl.BlockSpec((1, C), lambda b: (0, 0)),
                pl.BlockSpec((1, C), lambda b: (0, 0)),
                pl.BlockSpec((1, C), lambda b: (0, 0)),
                pl.BlockSpec((1, 1), lambda b: (0, 0)),
            ],
            out_specs=pl.BlockSpec((1, 1, K), lambda b: (b, 0, 0)),
            out_shape=jax.ShapeDtypeStruct((BH, 1, K), jnp.int32),
        )(x, ln_gamma.reshape(1, C), ln_beta.reshape(1, C),
          conf_w.reshape(1, C), conf_b.reshape(1, 1))

        nrows = BH * K
        rows_per = nrows // NW
        y_h, posd_h = pl.kernel(
            functools.partial(_sc_body, nrows=nrows),
            out_type=(jax.ShapeDtypeStruct((nrows, C), jnp.float32),
                      jax.ShapeDtypeStruct((nrows,), jnp.int32)),
            mesh=mesh,
            scratch_types=[
                pltpu.VMEM((rows_per,), jnp.int32),
                pltpu.VMEM((rows_per,), jnp.int32),
                pltpu.VMEM((_SC_CH, C), jnp.float32),
                pltpu.VMEM((_SC_CH, C), jnp.float32),
                pltpu.VMEM((_SC_CH, C), jnp.float32),
                pltpu.VMEM((_SC_CH, C), jnp.float32),
                pltpu.SemaphoreType.DMA,
                pltpu.SemaphoreType.DMA,
                pltpu.SemaphoreType.DMA,
                pltpu.SemaphoreType.DMA,
            ],
        )(gidx.reshape(nrows), x2d, pos_flat, pe)
        ys.append(y_h)
        ps.append(posd_h)

    y = jnp.concatenate(ys, axis=0)
    posd = jnp.concatenate(ps, axis=0)
    return y.reshape(B, K, C), posd.reshape(B, K)


# speedup vs baseline: 1.0626x; 1.0626x over previous
import functools

import jax
import jax.numpy as jnp
import numpy as np
from jax import lax
from jax.experimental import pallas as pl
from jax.experimental.pallas import tpu as pltpu
from jax.experimental.pallas import tpu_sc as plsc

K = 256
NC, NS = 2, 16
NW = NC * NS


_INV_C = float(np.float32(1.0 / 768.0))


def _col_sum_768(v):
    total = None
    for w in range(3):
        p = v[256 * w: 256 * w + 128, :] + v[256 * w + 128: 256 * w + 256, :]
        t = p[0:8, :]
        for j in range(1, 16):
            t = t + p[8 * j: 8 * j + 8, :]
        u = t[0:4, :] + t[4:8, :]
        q = u[0:2, :] + u[2:4, :]
        r = q[0:1, :] + q[1:2, :]
        total = r if w == 0 else total + r
    return total


def _topk_body(x_ref, g_ref, b_ref, w_ref, cb_ref, gidx_ref, *, bid_off=0):
    n = x_ref.shape[1]
    c = x_ref.shape[2]
    bid = pl.program_id(0) + bid_off
    xt = jnp.transpose(x_ref[0])
    mu = _col_sum_768(xt) * _INV_C
    xc = xt - mu
    var = _col_sum_768(xc * xc) * _INV_C
    g_col = jnp.transpose(g_ref[...])
    b_col = jnp.transpose(b_ref[...])
    ln = xc / jnp.sqrt(var + 1e-5) * g_col + b_col
    wrep = jnp.broadcast_to(jnp.transpose(w_ref[...]), (c, 128))
    conf = lax.dot_general(ln.astype(jnp.bfloat16), wrep.astype(jnp.bfloat16),
                           (((0,), (0,)), ((), ())),
                           preferred_element_type=jnp.float32)[:, 0:1] + cb_ref[0, 0]
    m = jnp.max(conf)
    e = jnp.exp(conf - m)
    s_col = e / jnp.sum(e) * float(n)
    ii = lax.broadcasted_iota(jnp.int32, (n, n), 0)
    jj = lax.broadcasted_iota(jnp.int32, (n, n), 1)
    s_row = jnp.transpose(s_col)
    beats = (s_row > s_col) | ((s_row == s_col) & (jj < ii))
    rank = jnp.sum(beats.astype(jnp.int32), axis=1, keepdims=True)
    kk = lax.broadcasted_iota(jnp.int32, (n, K), 1)
    oh = rank == kk
    rows = lax.broadcasted_iota(jnp.int32, (n, K), 0)
    idx = jnp.sum(jnp.where(oh, rows, 0), axis=0, keepdims=True)
    gidx_ref[0] = idx + bid * n


_SC_CH = 32


def _sc_body(gidx_hbm, x_hbm, pos_hbm, pe_hbm, y_hbm, posd_hbm,
             idx_v, posd_v, ra0, rb0, ra1, rb1, sem_a, sem_b, sem_s0, sem_s1,
             *, nrows):
    wid = lax.axis_index("s") * NC + lax.axis_index("c")
    rows_per = nrows // NW
    base = wid * rows_per
    pltpu.sync_copy(gidx_hbm.at[pl.ds(base, rows_per)], idx_v)
    pltpu.async_copy(pos_hbm.at[idx_v], posd_v, sem_a).wait()
    pltpu.sync_copy(posd_v, posd_hbm.at[pl.ds(base, rows_per)])
    ra = (ra0, ra1)
    rb = (rb0, rb1)
    ss = (sem_s0, sem_s1)
    nch = rows_per // _SC_CH
    gathers = {}

    def _issue(c2):
        p2 = c2 & 1
        ia = idx_v.at[pl.ds(c2 * _SC_CH, _SC_CH)]
        pa = posd_v.at[pl.ds(c2 * _SC_CH, _SC_CH)]
        gathers[c2] = (pltpu.async_copy(x_hbm.at[ia], ra[p2], sem_a),
                       pltpu.async_copy(pe_hbm.at[pa], rb[p2], sem_b))

    scats = {}
    _issue(0)
    for cch in range(nch):
        p = cch & 1
        ga, gb = gathers.pop(cch)
        ga.wait()
        gb.wait()
        if cch + 1 < nch:
            if cch - 1 >= 0:
                scats.pop(cch - 1).wait()
            _issue(cch + 1)

        def _add_row(r, _, _p=p):
            for kk in range(768 // 16):
                sl = pl.ds(kk * 16, 16)
                ra[_p][r, sl] = ra[_p][r, sl] + rb[_p][r, sl]
            return 0

        lax.fori_loop(0, _SC_CH, _add_row, 0)
        scats[cch] = pltpu.async_copy(
            ra[p], y_hbm.at[pl.ds(base + cch * _SC_CH, _SC_CH)], ss[p])
    scats.pop(nch - 2).wait()
    scats.pop(nch - 1).wait()


def kernel(x, pos, pos_embed, H, W, ln_gamma, ln_beta, conf_w, conf_b):
    B, N, C = x.shape
    L = pos_embed.shape[1]
    x2d = x.reshape(B * N, C)
    pos_flat = pos.reshape(B * N)
    pe = pos_embed.reshape(L, C)
    mesh = plsc.VectorSubcoreMesh(core_axis_name="c", subcore_axis_name="s",
                                  num_cores=NC, num_subcores=NS)
    gidx = pl.pallas_call(
        _topk_body,
        grid=(B,),
        in_specs=[
            pl.BlockSpec((1, N, C), lambda b: (b, 0, 0)),
            pl.BlockSpec((1, C), lambda b: (0, 0)),
            pl.BlockSpec((1, C), lambda b: (0, 0)),
            pl.BlockSpec((1, C), lambda b: (0, 0)),
            pl.BlockSpec((1, 1), lambda b: (0, 0)),
        ],
        out_specs=pl.BlockSpec((1, 1, K), lambda b: (b, 0, 0)),
        out_shape=jax.ShapeDtypeStruct((B, 1, K), jnp.int32),
    )(x, ln_gamma.reshape(1, C), ln_beta.reshape(1, C),
      conf_w.reshape(1, C), conf_b.reshape(1, 1))

    nrows = B * K
    rows_per = nrows // NW
    y, posd = pl.kernel(
        functools.partial(_sc_body, nrows=nrows),
        out_type=(jax.ShapeDtypeStruct((nrows, C), jnp.float32),
                  jax.ShapeDtypeStruct((nrows,), jnp.int32)),
        mesh=mesh,
        scratch_types=[
            pltpu.VMEM((rows_per,), jnp.int32),
            pltpu.VMEM((rows_per,), jnp.int32),
            pltpu.VMEM((_SC_CH, C), jnp.float32),
            pltpu.VMEM((_SC_CH, C), jnp.float32),
            pltpu.VMEM((_SC_CH, C), jnp.float32),
            pltpu.VMEM((_SC_CH, C), jnp.float32),
            pltpu.SemaphoreType.DMA,
            pltpu.SemaphoreType.DMA,
            pltpu.SemaphoreType.DMA,
            pltpu.SemaphoreType.DMA,
        ],
    )(gidx.reshape(nrows), x2d, pos_flat, pe)

    return y.reshape(B, K, C), posd.reshape(B, K)
